# baseline (speedup 1.0000x reference)
import jax
import jax.numpy as jnp
from jax import lax
from jax.experimental import pallas as pl
from jax.experimental.pallas import tpu as pltpu
from jax.experimental.pallas import tpu_sc as plsc

_B = 16384
_F = 26
_FIELD = 40000
_NT = 16
_NPAIR = _F // 2
_BPC = _B // 2
_L = 16
_BPT = _BPC // _NT
_NJV = _BPC // _L
_TW = 128
_RPP = 2 * _FIELD // _TW
_NPAD = 1040384


def _sc_body(x_hbm, table_hbm, bias_hbm, out_hbm,
             tblk, xab, partial, red, outv, biasv, shared, sem):
  nc = lax.axis_index("c")
  ns = lax.axis_index("s")
  cbase = nc * _BPC

  @pl.when(ns < _NPAIR)
  def gather_phase():
    fa = 2 * ns
    row_lo = ns * _RPP
    cps = [
        pltpu.async_copy(table_hbm.at[pl.ds(row_lo, _RPP), :], tblk, sem),
        pltpu.async_copy(
            x_hbm.at[pl.ds(fa * _B + cbase, _BPC)], xab.at[pl.ds(0, _BPC)],
            sem),
        pltpu.async_copy(
            x_hbm.at[pl.ds((fa + 1) * _B + cbase, _BPC)],
            xab.at[pl.ds(_BPC, _BPC)], sem),
    ]
    for cp in cps:
      cp.wait()

    def body(j, _):
      sl = pl.ds(j * _L, _L)
      ia = xab[pl.ds(j * _L, _L)]
      ib = xab[pl.ds(_BPC + j * _L, _L)] + _FIELD
      va = plsc.load_gather(tblk, [ia >> 7, ia & 127])
      vb = plsc.load_gather(tblk, [ib >> 7, ib & 127])
      partial[sl] = va + vb
      return 0

    lax.fori_loop(0, _NJV, body, 0)
    pltpu.sync_copy(partial, shared.at[pl.ds(ns * _BPC, _BPC)])

  plsc.subcore_barrier()

  pltpu.sync_copy(bias_hbm, biasv)
  cps = [
      pltpu.async_copy(
          shared.at[pl.ds(t * _BPC + ns * _BPT, _BPT)],
          red.at[pl.ds(t * _BPT, _BPT)],
          sem,
      )
      for t in range(_NPAIR)
  ]
  for cp in cps:
    cp.wait()
  bvec = biasv[pl.ds(0, _L)]

  def reduce(jv, _):
    sl = pl.ds(jv * _L, _L)
    j16 = jv * _L
    acc = bvec + red[pl.ds(j16, _L)]
    for t in range(1, _NPAIR):
      acc = acc + red[pl.ds(t * _BPT + j16, _L)]
    outv[sl] = acc
    return 0

  lax.fori_loop(0, _BPT // _L, reduce, 0)
  pltpu.sync_copy(outv, out_hbm.at[pl.ds(cbase + ns * _BPT, _BPT)])


@jax.jit
def kernel(x, table, bias):
  xt = x.astype(jnp.int32).T.reshape(-1)
  tpad = jnp.pad(table.astype(jnp.float32), ((0, _NPAD - _F * _FIELD), (0, 0)))
  t2 = tpad.reshape(-1).reshape(_NPAD // _TW, _TW)

  mesh = plsc.VectorSubcoreMesh(
      core_axis_name="c", subcore_axis_name="s", num_cores=2, num_subcores=16)
  out = pl.kernel(
      _sc_body,
      out_type=jax.ShapeDtypeStruct((_B,), jnp.float32),
      mesh=mesh,
      compiler_params=pltpu.CompilerParams(
          needs_layout_passes=False, use_tc_tiling_on_sc=False),
      scratch_types=[
          pltpu.VMEM((_RPP, _TW), jnp.float32),
          pltpu.VMEM((2 * _BPC,), jnp.int32),
          pltpu.VMEM((_BPC,), jnp.float32),
          pltpu.VMEM((_NPAIR * _BPT,), jnp.float32),
          pltpu.VMEM((_BPT,), jnp.float32),
          pltpu.VMEM((_L,), jnp.float32),
          pltpu.VMEM_SHARED((_NPAIR * _BPC,), jnp.float32),
          pltpu.SemaphoreType.DMA,
      ],
  )(xt, t2, jnp.broadcast_to(bias.astype(jnp.float32), (_L,)))
  return out.reshape(_B, 1)
